# Initial kernel scaffold; baseline (speedup 1.0000x reference)
import jax
import jax.numpy as jnp
from jax.experimental import pallas as pl


def kernel(shortest_path, edge_feat, graph_attn_bias, W_edge, W_sp, W_vnode):
    raise NotImplementedError("write your pallas kernel here")


# speedup vs baseline: 6.0100x; 6.0100x over previous
import functools

import jax
import jax.numpy as jnp
from jax import lax
from jax.experimental import pallas as pl
from jax.experimental.pallas import tpu as pltpu
from jax.experimental.pallas import tpu_sc as plsc

PAIR_DIM = 32
B = 32
N = 255
NP1 = 256
NUM_CORES = 2
NUM_SUBCORES = 16
L = 16


def _sc_body(sp_idx, ed_idx, w_edge, w_sp, w_vnode, out,
             tsp, ted, vno, spbuf, edbuf, blk, vplane):
    b = lax.axis_index("s") * NUM_CORES + lax.axis_index("c")

    pltpu.sync_copy(w_sp, tsp)
    pltpu.sync_copy(w_edge, ted)
    pltpu.sync_copy(w_vnode, vno)

    v0 = vno[0, pl.ds(0, L)]
    v1 = vno[0, pl.ds(L, L)]

    def fill(j, _):
        vplane[j, pl.ds(0, L)] = v0
        vplane[j, pl.ds(L, L)] = v1
        return 0

    lax.fori_loop(0, NP1, fill, 0)
    pltpu.sync_copy(vplane, out.at[b, 0])

    third = jnp.float32(1.0 / 3.0)
    iota = lax.iota(jnp.int32, L)

    def unit(i, _):
        u = b * N + i
        pltpu.sync_copy(sp_idx.at[u], spbuf)
        pltpu.sync_copy(ed_idx.at[u], edbuf)

        def group(g, _):
            j0 = g * L
            rows = j0 + iota
            spr = spbuf[pl.ds(j0, L)]
            e0 = edbuf[0, pl.ds(j0, L)]
            e1 = edbuf[1, pl.ds(j0, L)]
            e2 = edbuf[2, pl.ds(j0, L)]
            for c in range(PAIR_DIM):
                cc = jnp.full((L,), c, jnp.int32)
                acc = plsc.load_gather(tsp, [spr, cc]) + third * (
                    plsc.load_gather(ted, [e0, cc])
                    + plsc.load_gather(ted, [e1, cc])
                    + plsc.load_gather(ted, [e2, cc])
                )
                plsc.store_scatter(blk, [rows, cc], acc)
            return 0

        lax.fori_loop(0, NP1 // L, group, 0)
        blk[0, pl.ds(0, L)] = v0
        blk[0, pl.ds(L, L)] = v1
        pltpu.sync_copy(blk, out.at[b, i + 1])
        return 0

    lax.fori_loop(0, N, unit, 0)


@jax.jit
def kernel(shortest_path, edge_feat, graph_attn_bias, W_edge, W_sp, W_vnode):
    del graph_attn_bias
    sp_idx = jnp.pad(shortest_path.reshape(B * N, N), ((0, 0), (1, 0)))
    ed_idx = jnp.pad(
        edge_feat.reshape(B * N, N, 3).transpose(0, 2, 1), ((0, 0), (0, 0), (1, 0))
    )

    mesh = plsc.VectorSubcoreMesh(
        core_axis_name="c", subcore_axis_name="s",
        num_cores=NUM_CORES, num_subcores=NUM_SUBCORES,
    )
    run = pl.kernel(
        _sc_body,
        out_type=jax.ShapeDtypeStruct((B, NP1, NP1, PAIR_DIM), jnp.float32),
        mesh=mesh,
        compiler_params=pltpu.CompilerParams(
            needs_layout_passes=False, use_tc_tiling_on_sc=False
        ),
        scratch_types=[
            pltpu.VMEM((512, PAIR_DIM), jnp.float32),
            pltpu.VMEM((1024, PAIR_DIM), jnp.float32),
            pltpu.VMEM((1, PAIR_DIM), jnp.float32),
            pltpu.VMEM((NP1,), jnp.int32),
            pltpu.VMEM((3, NP1), jnp.int32),
            pltpu.VMEM((NP1, PAIR_DIM), jnp.float32),
            pltpu.VMEM((NP1, PAIR_DIM), jnp.float32),
        ],
    )
    return run(sp_idx, ed_idx, W_edge, W_sp, W_vnode)
